# baseline (speedup 1.0000x reference)
import jax
import jax.numpy as jnp
from jax.experimental import pallas as pl


def _norm_body(x_ref, o_ref):
    xb = x_ref[...]
    s = jnp.sum(xb * xb, axis=1, keepdims=True)
    r = jax.lax.rsqrt(jnp.maximum(s, 1e-24))
    o_ref[...] = xb * r


def kernel(x):
    M, D = x.shape
    BM = 512
    return pl.pallas_call(
        _norm_body,
        grid=(M // BM,),
        in_specs=[pl.BlockSpec((BM, D), lambda i: (i, 0))],
        out_specs=pl.BlockSpec((BM, D), lambda i: (i, 0)),
        out_shape=jax.ShapeDtypeStruct((M, D), x.dtype),
    )(x)


# speedup vs baseline: 1.7150x; 1.7150x over previous
import jax
import jax.numpy as jnp
from jax.experimental import pallas as pl


def _norm_body(x_ref, o_ref):
    xb = x_ref[...]
    s = jnp.sum(xb * xb, axis=1, keepdims=True)
    r = jax.lax.rsqrt(jnp.maximum(s, 1e-24))
    o_ref[...] = xb * r


def kernel(x):
    M, D = x.shape
    BM = 2048
    return pl.pallas_call(
        _norm_body,
        grid=(M // BM,),
        in_specs=[pl.BlockSpec((BM, D), lambda i: (i, 0))],
        out_specs=pl.BlockSpec((BM, D), lambda i: (i, 0)),
        out_shape=jax.ShapeDtypeStruct((M, D), x.dtype),
    )(x)


# speedup vs baseline: 1.8122x; 1.0567x over previous
import jax
import jax.numpy as jnp
from jax.experimental import pallas as pl


def _norm_body(x_ref, o_ref):
    xb = x_ref[...]
    s = jnp.sum(xb * xb, axis=1, keepdims=True)
    r = jax.lax.rsqrt(jnp.maximum(s, 1e-24))
    o_ref[...] = xb * r


def kernel(x):
    M, D = x.shape
    BM = 4096
    return pl.pallas_call(
        _norm_body,
        grid=(M // BM,),
        in_specs=[pl.BlockSpec((BM, D), lambda i: (i, 0))],
        out_specs=pl.BlockSpec((BM, D), lambda i: (i, 0)),
        out_shape=jax.ShapeDtypeStruct((M, D), x.dtype),
    )(x)


# speedup vs baseline: 1.9109x; 1.0545x over previous
import jax
import jax.numpy as jnp
from jax.experimental import pallas as pl


def _norm_body(x_ref, o_ref):
    xb = x_ref[...]
    s = jnp.sum(xb * xb, axis=1, keepdims=True)
    r = jax.lax.rsqrt(jnp.maximum(s, 1e-24))
    o_ref[...] = xb * r


def kernel(x):
    M, D = x.shape
    BM = 8192
    return pl.pallas_call(
        _norm_body,
        grid=(M // BM,),
        in_specs=[pl.BlockSpec((BM, D), lambda i: (i, 0))],
        out_specs=pl.BlockSpec((BM, D), lambda i: (i, 0)),
        out_shape=jax.ShapeDtypeStruct((M, D), x.dtype),
    )(x)
